# Initial kernel scaffold; baseline (speedup 1.0000x reference)
import jax
import jax.numpy as jnp
from jax.experimental import pallas as pl


def kernel(x, edge_index, batch, W1r, b1, W1root, p1, W2r, b2, W2root, p2, W3r, b3, W3root, p3, L1W, L1b, L2W, L2b, L3W, L3b):
    raise NotImplementedError("write your pallas kernel here")


# speedup vs baseline: 14.1551x; 14.1551x over previous
import functools

import jax
import jax.numpy as jnp
from jax import lax
from jax.experimental import pallas as pl
from jax.experimental.pallas import tpu as pltpu
from jax.experimental.pallas import tpu_sc as plsc

N = 10000
E = 320000
D = 128
KS = (5000, 2500, 1250)

NC = 2
NS = 16
NW = NC * NS
EPW = E // NW
CH = 128
NFULL = EPW // CH
REM = EPW - NFULL * CH
STRIPE = 624

NPAD = 10240
SROWS = NPAD // 128
RB = 2000
GA = N // RB
IMIN = -2147483648


def _sc_scatter_body(x_hbm, src_hbm, dst_hbm, out_hbm,
                     srcv, dstv, rows, srcr, dstr, rowsr, acc, sem):
    cid = lax.axis_index("c")
    sid = lax.axis_index("s")
    wid = sid * NC + cid

    def zrow(i, carry):
        for j in range(8):
            rows[i, pl.ds(j * 16, 16)] = jnp.zeros((16,), jnp.float32)
        return carry
    lax.fori_loop(0, CH, zrow, 0)
    base = sid * STRIPE
    for t in range(4):
        pltpu.sync_copy(rows.at[pl.ds(0, CH)], acc.at[pl.ds(base + t * CH, CH)])
    pltpu.sync_copy(rows.at[pl.ds(0, STRIPE - 4 * CH)],
                    acc.at[pl.ds(base + 4 * CH, STRIPE - 4 * CH)])

    @pl.when(sid == NS - 1)
    def _():
        pltpu.sync_copy(rows.at[pl.ds(0, N - NS * STRIPE)],
                        acc.at[pl.ds(NS * STRIPE, N - NS * STRIPE)])
    plsc.subcore_barrier()

    ebase = wid * EPW

    def chunk(i, carry):
        b = ebase + i * CH
        pltpu.sync_copy(src_hbm.at[pl.ds(b, CH)], srcv)
        pltpu.sync_copy(dst_hbm.at[pl.ds(b, CH)], dstv)
        pltpu.async_copy(x_hbm.at[srcv], rows, sem).wait()
        pltpu.sync_copy(rows, acc.at[dstv], add=True)
        return carry
    lax.fori_loop(0, NFULL, chunk, 0)

    b = ebase + NFULL * CH
    pltpu.sync_copy(src_hbm.at[pl.ds(b, REM)], srcr)
    pltpu.sync_copy(dst_hbm.at[pl.ds(b, REM)], dstr)
    pltpu.async_copy(x_hbm.at[srcr], rowsr, sem).wait()
    pltpu.sync_copy(rowsr, acc.at[dstr], add=True)

    plsc.subcore_barrier()
    pltpu.sync_copy(acc.at[pl.ds(base, STRIPE)],
                    out_hbm.at[cid, pl.ds(base, STRIPE)])

    @pl.when(sid == NS - 1)
    def _():
        pltpu.sync_copy(acc.at[pl.ds(NS * STRIPE, N - NS * STRIPE)],
                        out_hbm.at[cid, pl.ds(NS * STRIPE, N - NS * STRIPE)])


_sc_scatter = functools.partial(
    pl.kernel,
    out_type=jax.ShapeDtypeStruct((NC, N, D), jnp.float32),
    mesh=plsc.VectorSubcoreMesh(core_axis_name="c", subcore_axis_name="s"),
    scratch_types=[
        pltpu.VMEM((CH,), jnp.int32),
        pltpu.VMEM((CH,), jnp.int32),
        pltpu.VMEM((CH, D), jnp.float32),
        pltpu.VMEM((REM,), jnp.int32),
        pltpu.VMEM((REM,), jnp.int32),
        pltpu.VMEM((REM, D), jnp.float32),
        pltpu.VMEM_SHARED((N, D), jnp.float32),
        pltpu.SemaphoreType.DMA,
    ],
)(_sc_scatter_body)


def _dense_body(aggp_ref, xm_ref, wr_ref, wroot_ref, b_ref, p_ref, h_ref, s_ref):
    aggp = aggp_ref[...]
    acc = aggp[0] + aggp[1]
    pre = (jnp.dot(acc, wr_ref[...], preferred_element_type=jnp.float32)
           + jnp.dot(xm_ref[...], wroot_ref[...], preferred_element_type=jnp.float32)
           + b_ref[...])
    h = jnp.maximum(pre, 0.0)
    p = p_ref[...]
    nrm = jnp.sqrt(jnp.sum(p * p))
    s = jnp.tanh(jnp.dot(h, p, preferred_element_type=jnp.float32) / nrm)
    h_ref[...] = h
    s_ref[...] = s


def _dense(parts, xm, wr, wroot, bb, p):
    return pl.pallas_call(
        _dense_body,
        grid=(GA,),
        in_specs=[
            pl.BlockSpec((NC, RB, D), lambda i: (0, i, 0)),
            pl.BlockSpec((RB, D), lambda i: (i, 0)),
            pl.BlockSpec((D, D), lambda i: (0, 0)),
            pl.BlockSpec((D, D), lambda i: (0, 0)),
            pl.BlockSpec((1, D), lambda i: (0, 0)),
            pl.BlockSpec((D, 1), lambda i: (0, 0)),
        ],
        out_specs=[pl.BlockSpec((RB, D), lambda i: (i, 0)),
                   pl.BlockSpec((RB, 1), lambda i: (i, 0))],
        out_shape=[jax.ShapeDtypeStruct((N, D), jnp.float32),
                   jax.ShapeDtypeStruct((N, 1), jnp.float32)],
    )(parts, xm, wr, wroot, bb, p)


def _select_body(k, nprev, score_ref, mask_ref, *refs):
    prev_refs = refs[:nprev]
    selw_ref, nmask_ref, skey_ref = refs[nprev:]
    s = score_ref[...]
    m = mask_ref[...]
    ibits = lax.bitcast_convert_type(s, jnp.int32)
    skey = jnp.where(ibits < 0, ibits ^ jnp.int32(0x7FFFFFFF), ibits)
    r = lax.broadcasted_iota(jnp.int32, (SROWS, 128), 0)
    c = lax.broadcasted_iota(jnp.int32, (SROWS, 128), 1)
    gidx = r * 128 + c
    valid = (m > 0) & (gidx < N)

    eq = valid
    need = jnp.int32(k)
    sel = jnp.zeros_like(valid)
    keys = [skey] + [pr[...] for pr in prev_refs]
    for key_full in keys:
        key = jnp.where(eq, key_full, jnp.int32(IMIN))

        def tbit(i, pu):
            bb = 31 - i
            trial = pu | (jnp.int32(1) << bb)
            thr = trial ^ jnp.int32(IMIN)
            cnt = jnp.sum((key >= thr).astype(jnp.int32))
            return jnp.where(cnt >= need, trial, pu)
        pu = lax.fori_loop(0, 32, tbit, jnp.int32(0))
        t = pu ^ jnp.int32(IMIN)
        gt = eq & (key > t)
        sel = sel | gt
        need = need - jnp.sum(gt.astype(jnp.int32))
        eq = eq & (key == t)

    def jbit(i, jj):
        bb = 13 - i
        trial = jj | (jnp.int32(1) << bb)
        g = jnp.sum((eq & (gidx < trial)).astype(jnp.int32))
        return jnp.where(g < need, trial, jj)
    jmax = lax.fori_loop(0, 14, jbit, jnp.int32(0))
    sel = sel | (eq & (gidx <= jmax) & (need > 0))

    nm = sel.astype(jnp.float32)
    nmask_ref[...] = nm
    selw_ref[...] = s * nm
    skey_ref[...] = skey


def _select(k, score2d, mask2d, prev_skeys):
    nprev = len(prev_skeys)
    return pl.pallas_call(
        functools.partial(_select_body, k, nprev),
        out_shape=[jax.ShapeDtypeStruct((SROWS, 128), jnp.float32),
                   jax.ShapeDtypeStruct((SROWS, 128), jnp.float32),
                   jax.ShapeDtypeStruct((SROWS, 128), jnp.int32)],
    )(score2d, mask2d, *prev_skeys)


def _finalize_body(h_ref, selw_ref, nm_ref, y_ref, rmax_ref, rsum_ref):
    i = pl.program_id(0)
    h = h_ref[...]
    w = selw_ref[...]
    m = nm_ref[...]
    y = h * w
    y_ref[...] = y
    masked = jnp.where(m > 0, y, -jnp.inf)
    bmax = jnp.max(masked, axis=0, keepdims=True)
    bsum = jnp.sum(y, axis=0, keepdims=True)

    @pl.when(i == 0)
    def _():
        rmax_ref[...] = bmax
        rsum_ref[...] = bsum

    @pl.when(i != 0)
    def _():
        rmax_ref[...] = jnp.maximum(rmax_ref[...], bmax)
        rsum_ref[...] = rsum_ref[...] + bsum


def _finalize(h, selw, nm):
    return pl.pallas_call(
        _finalize_body,
        grid=(GA,),
        in_specs=[pl.BlockSpec((RB, D), lambda i: (i, 0)),
                  pl.BlockSpec((RB, 1), lambda i: (i, 0)),
                  pl.BlockSpec((RB, 1), lambda i: (i, 0))],
        out_specs=[pl.BlockSpec((RB, D), lambda i: (i, 0)),
                   pl.BlockSpec((1, D), lambda i: (0, 0)),
                   pl.BlockSpec((1, D), lambda i: (0, 0))],
        out_shape=[jax.ShapeDtypeStruct((N, D), jnp.float32),
                   jax.ShapeDtypeStruct((1, D), jnp.float32),
                   jax.ShapeDtypeStruct((1, D), jnp.float32)],
    )(h, selw, nm)


def _head_body(mx1, sm1, mx2, sm2, mx3, sm3, wa, wb, b1, w2, b2, w3, b3, out):
    zmax = mx1[...] + mx2[...] + mx3[...]
    zmean = sm1[...] / KS[0] + sm2[...] / KS[1] + sm3[...] / KS[2]
    a = jnp.maximum(jnp.dot(zmax, wa[...], preferred_element_type=jnp.float32)
                    + jnp.dot(zmean, wb[...], preferred_element_type=jnp.float32)
                    + b1[...], 0.0)
    a = jnp.maximum(jnp.dot(a, w2[...], preferred_element_type=jnp.float32)
                    + b2[...], 0.0)
    lg = jnp.dot(a, w3[...], preferred_element_type=jnp.float32) + b3[...]
    mx = jnp.max(lg, axis=1, keepdims=True)
    e = jnp.exp(lg - mx)
    out[...] = lg - mx - jnp.log(jnp.sum(e, axis=1, keepdims=True))


def _head(reads, L1W, L1b, L2W, L2b, L3W, L3b):
    args = []
    for rmax, rsum in reads:
        args += [rmax, rsum]
    args += [L1W[:D], L1W[D:], L1b.reshape(1, -1), L2W, L2b.reshape(1, -1),
             L3W, L3b.reshape(1, -1)]
    return pl.pallas_call(
        _head_body,
        out_shape=jax.ShapeDtypeStruct((1, 16), jnp.float32),
    )(*args)


def kernel(x, edge_index, batch, W1r, b1, W1root, p1, W2r, b2, W2root, p2,
           W3r, b3, W3root, p3, L1W, L1b, L2W, L2b, L3W, L3b):
    src = edge_index[0]
    dst = edge_index[1]
    Ws = ((W1r, b1, W1root, p1), (W2r, b2, W2root, p2), (W3r, b3, W3root, p3))

    xm = x
    mask2d = jnp.ones((SROWS, 128), jnp.float32)
    skeys = []
    reads = []
    for l in range(3):
        Wr, bb, Wroot, p = Ws[l]
        parts = _sc_scatter(xm, src, dst)
        h, score = _dense(parts, xm, Wr, Wroot, bb.reshape(1, D), p.reshape(D, 1))
        score2d = jnp.reshape(jnp.pad(score, ((0, NPAD - N), (0, 0))), (SROWS, 128))
        selw2d, mask2d, skey2d = _select(KS[l], score2d, mask2d, skeys)
        skeys.insert(0, skey2d)
        selw = jnp.reshape(selw2d, (NPAD, 1))[:N]
        nm = jnp.reshape(mask2d, (NPAD, 1))[:N]
        xm, rmax, rsum = _finalize(h, selw, nm)
        reads.append((rmax, rsum))

    return _head(reads, L1W, L1b, L2W, L2b, L3W, L3b)
